# Initial kernel scaffold; baseline (speedup 1.0000x reference)
import jax
import jax.numpy as jnp
from jax.experimental import pallas as pl


def kernel(spectrograms):
    raise NotImplementedError("write your pallas kernel here")


# speedup vs baseline: 1.5137x; 1.5137x over previous
import functools

import jax
import jax.numpy as jnp
from jax import lax
from jax.experimental import pallas as pl
from jax.experimental.pallas import tpu as pltpu
from jax.experimental.pallas import tpu_sc as plsc

MIN_STEPS = -12
MAX_STEPS = 12
LOWER_BIN = MAX_STEPS

B, C, H = 64, 512, 360
OUT = H - MAX_STEPS + MIN_STEPS
NGRP = OUT // 16
R = 64
NB = 2


def _pitch_shift_sc(spectrograms, first_bin):
    mesh = plsc.VectorSubcoreMesh(core_axis_name="c", subcore_axis_name="s")

    @functools.partial(
        pl.kernel,
        out_type=(
            jax.ShapeDtypeStruct((B, C, OUT), jnp.float32),
            jax.ShapeDtypeStruct((B, C, OUT), jnp.float32),
        ),
        mesh=mesh,
        scratch_types=[
            pltpu.VMEM((R, H), jnp.float32),
            pltpu.VMEM((R, OUT), jnp.float32),
            pltpu.VMEM((R, OUT), jnp.float32),
            pltpu.VMEM((16,), jnp.int32),
        ],
        compiler_params=pltpu.CompilerParams(use_tc_tiling_on_sc=False,
                                               needs_layout_passes=False),
    )
    def k(spec_hbm, fb_hbm, x_hbm, xt_hbm, buf, outx, outxt, fbv):
        cid = lax.axis_index("c")
        sid = lax.axis_index("s")
        wid = sid * 2 + cid
        lane = lax.iota(jnp.int32, 16)
        for bi in range(NB):
            b = wid * NB + bi
            blk = pl.multiple_of((b // 16) * 16, 16)
            pltpu.sync_copy(fb_hbm.at[pl.ds(blk, 16)], fbv)
            off = jnp.sum(jnp.where(lane == (b % 16), fbv[...], 0))
            for r0 in range(0, C, R):
                pltpu.sync_copy(spec_hbm.at[b, pl.ds(r0, R), :], buf)

                def row_body(r, _):
                    rvec = jnp.full((16,), r, dtype=jnp.int32)
                    for g in range(NGRP):
                        colx = LOWER_BIN + g * 16 + lane
                        colt = off + g * 16 + lane
                        vx = plsc.load_gather(buf, [rvec, colx])
                        vt = plsc.load_gather(buf, [rvec, colt])
                        outx.at[r][pl.ds(g * 16, 16)] = vx
                        outxt.at[r][pl.ds(g * 16, 16)] = vt
                    return _

                lax.fori_loop(0, R, row_body, None)
                pltpu.sync_copy(outx, x_hbm.at[b, pl.ds(r0, R), :])
                pltpu.sync_copy(outxt, xt_hbm.at[b, pl.ds(r0, R), :])

    return k(spectrograms, first_bin)


def kernel(spectrograms):
    batch_size = spectrograms.shape[0]
    k = jax.random.fold_in(jax.random.key(0), 1)
    n_steps = jax.random.randint(k, (batch_size,), MIN_STEPS, MAX_STEPS + 1,
                                 dtype=jnp.int32)
    first_bin = (LOWER_BIN - n_steps).astype(jnp.int32)
    x, xt = _pitch_shift_sc(spectrograms, first_bin)
    return (x, xt, n_steps)


# speedup vs baseline: 2.2552x; 1.4899x over previous
import functools

import jax
import jax.numpy as jnp
from jax import lax
from jax.experimental import pallas as pl
from jax.experimental.pallas import tpu as pltpu
from jax.experimental.pallas import tpu_sc as plsc

MIN_STEPS = -12
MAX_STEPS = 12
LOWER_BIN = MAX_STEPS

B, C, H = 64, 512, 360
OUT = H - MAX_STEPS + MIN_STEPS
NGRP = OUT // 16
R = 64
NB = 2


def _pitch_shift_sc(spectrograms, first_bin):
    mesh = plsc.VectorSubcoreMesh(core_axis_name="c", subcore_axis_name="s")

    @functools.partial(
        pl.kernel,
        out_type=(
            jax.ShapeDtypeStruct((B, C, OUT), jnp.float32),
            jax.ShapeDtypeStruct((B, C, OUT), jnp.float32),
        ),
        mesh=mesh,
        scratch_types=[
            pltpu.VMEM((R, H), jnp.float32),
            pltpu.VMEM((R, OUT), jnp.float32),
            pltpu.VMEM((R, OUT), jnp.float32),
            pltpu.VMEM((16,), jnp.int32),
        ],
        compiler_params=pltpu.CompilerParams(needs_layout_passes=False),
    )
    def k(spec_hbm, fb_hbm, x_hbm, xt_hbm, buf, outx, outxt, fbv):
        cid = lax.axis_index("c")
        sid = lax.axis_index("s")
        wid = sid * 2 + cid
        lane = lax.iota(jnp.int32, 16)
        for bi in range(NB):
            b = wid * NB + bi
            blk = pl.multiple_of((b // 16) * 16, 16)
            pltpu.sync_copy(fb_hbm.at[pl.ds(blk, 16)], fbv)
            off = jnp.sum(jnp.where(lane == (b % 16), fbv[...], 0))
            colt = [off + g * 16 + lane for g in range(NGRP)]
            colx = [LOWER_BIN + g * 16 + lane for g in range(NGRP)]
            for r0 in range(0, C, R):
                pltpu.sync_copy(spec_hbm.at[b, pl.ds(r0, R), :], buf)

                def row_body(r, _):
                    rvec = jnp.full((16,), r, dtype=jnp.int32)
                    for g in range(NGRP):
                        c0 = g * 16
                        vx = plsc.load_gather(buf, [rvec, colx[g]])
                        vt = plsc.load_gather(buf, [rvec, colt[g]])
                        outx[r, pl.ds(c0, 16)] = vx
                        outxt[r, pl.ds(c0, 16)] = vt
                    return _

                lax.fori_loop(0, R, row_body, None)
                pltpu.sync_copy(outx, x_hbm.at[b, pl.ds(r0, R), :])
                pltpu.sync_copy(outxt, xt_hbm.at[b, pl.ds(r0, R), :])

    return k(spectrograms, first_bin)


def kernel(spectrograms):
    batch_size = spectrograms.shape[0]
    k = jax.random.fold_in(jax.random.key(0), 1)
    n_steps = jax.random.randint(k, (batch_size,), MIN_STEPS, MAX_STEPS + 1,
                                 dtype=jnp.int32)
    first_bin = (LOWER_BIN - n_steps).astype(jnp.int32)
    x, xt = _pitch_shift_sc(spectrograms, first_bin)
    return (x, xt, n_steps)


# speedup vs baseline: 2.2581x; 1.0013x over previous
import functools

import jax
import jax.numpy as jnp
from jax import lax
from jax.experimental import pallas as pl
from jax.experimental.pallas import tpu as pltpu
from jax.experimental.pallas import tpu_sc as plsc

MIN_STEPS = -12
MAX_STEPS = 12
LOWER_BIN = MAX_STEPS

B, C, H = 64, 512, 360
OUT = H - MAX_STEPS + MIN_STEPS
NGRP = OUT // 16
R = 64
NB = 2


def _pitch_shift_sc(spectrograms, first_bin):
    mesh = plsc.VectorSubcoreMesh(
        core_axis_name="c", subcore_axis_name="s", num_cores=2)

    @functools.partial(
        pl.kernel,
        out_type=(
            jax.ShapeDtypeStruct((B, C, OUT), jnp.float32),
            jax.ShapeDtypeStruct((B, C, OUT), jnp.float32),
        ),
        mesh=mesh,
        scratch_types=[
            pltpu.VMEM((R, H), jnp.float32),
            pltpu.VMEM((R, OUT), jnp.float32),
            pltpu.VMEM((R, OUT), jnp.float32),
            pltpu.VMEM((16,), jnp.int32),
        ],
        compiler_params=pltpu.CompilerParams(needs_layout_passes=False),
    )
    def k(spec_hbm, fb_hbm, x_hbm, xt_hbm, buf, outx, outxt, fbv):
        cid = lax.axis_index("c")
        sid = lax.axis_index("s")
        wid = sid * 2 + cid
        lane = lax.iota(jnp.int32, 16)
        for bi in range(NB):
            b = wid * NB + bi
            blk = pl.multiple_of((b // 16) * 16, 16)
            pltpu.sync_copy(fb_hbm.at[pl.ds(blk, 16)], fbv)
            off = jnp.sum(jnp.where(lane == (b % 16), fbv[...], 0))
            colt = [off + g * 16 + lane for g in range(NGRP)]
            colx = [LOWER_BIN + g * 16 + lane for g in range(NGRP)]
            for r0 in range(0, C, R):
                pltpu.sync_copy(spec_hbm.at[b, pl.ds(r0, R), :], buf)

                def row_body(r, _):
                    rvec = jnp.full((16,), r, dtype=jnp.int32)
                    for g in range(NGRP):
                        c0 = g * 16
                        vx = plsc.load_gather(buf, [rvec, colx[g]])
                        vt = plsc.load_gather(buf, [rvec, colt[g]])
                        outx[r, pl.ds(c0, 16)] = vx
                        outxt[r, pl.ds(c0, 16)] = vt
                    return _

                lax.fori_loop(0, R, row_body, None)
                pltpu.sync_copy(outx, x_hbm.at[b, pl.ds(r0, R), :])
                pltpu.sync_copy(outxt, xt_hbm.at[b, pl.ds(r0, R), :])

    return k(spectrograms, first_bin)


def kernel(spectrograms):
    batch_size = spectrograms.shape[0]
    k = jax.random.fold_in(jax.random.key(0), 1)
    n_steps = jax.random.randint(k, (batch_size,), MIN_STEPS, MAX_STEPS + 1,
                                 dtype=jnp.int32)
    first_bin = (LOWER_BIN - n_steps).astype(jnp.int32)
    x, xt = _pitch_shift_sc(spectrograms, first_bin)
    return (x, xt, n_steps)
